# baseline (speedup 1.0000x reference)
import functools

import jax
import jax.numpy as jnp
from jax import lax
from jax.experimental import pallas as pl
from jax.experimental.pallas import tpu as pltpu
from jax.experimental.pallas import tpu_sc as plsc

D = 128
LANES = 16
NW = 32


def _gsum_body(table, idxb, out, idx_v, rows_v, acc_v, sem, *,
               n_chunks, n_iter, chunk, apply_relu, lanes):
    cid = lax.axis_index("c")
    sid = lax.axis_index("s")
    wid = sid * 2 + cid
    nv = D // lanes

    def chunk_body(t, carry):
        c = t * NW + wid

        @pl.when(c < n_chunks)
        def _():
            pltpu.sync_copy(idxb.at[c], idx_v)
            descs = [pltpu.async_copy(table.at[idx_v.at[j]], rows_v.at[j], sem)
                     for j in range(4)]
            for d in descs:
                d.wait()

            if lanes == LANES:
                def row_body(r, rc):
                    for k in range(nv):
                        s = pl.ds(k * LANES, LANES)
                        vs = [rows_v[j, r, s] for j in range(4)]
                        if apply_relu:
                            vs = [jnp.maximum(v, 0) for v in vs]
                        acc_v[r, s] = (vs[0] + vs[1]) + (vs[2] + vs[3])
                    return rc

                lax.fori_loop(0, chunk, row_body, 0)
            else:
                def row_body(r, rc):
                    base = pl.multiple_of(r * 2, 2)
                    rr = pl.ds(base, 2)
                    for k in range(D // LANES):
                        s = pl.ds(k * LANES, LANES)
                        vs = [rows_v[j, rr, s] for j in range(4)]
                        if apply_relu:
                            vs = [jnp.maximum(v, 0) for v in vs]
                        acc_v[rr, s] = (vs[0] + vs[1]) + (vs[2] + vs[3])
                    return rc

                lax.fori_loop(0, chunk // 2, row_body, 0)
            pltpu.sync_copy(acc_v, out.at[pl.ds(c * chunk, chunk)])

        return carry

    lax.fori_loop(0, n_iter, chunk_body, 0)


def _gsum_sc(table, idxb, n_out, chunk, apply_relu):
    n_chunks = n_out // chunk
    n_iter = (n_chunks + NW - 1) // NW
    lanes = 32 if table.dtype == jnp.bfloat16 else LANES
    mesh = plsc.VectorSubcoreMesh(core_axis_name="c", subcore_axis_name="s",
                                  num_cores=2, num_subcores=16)
    kern = pl.kernel(
        functools.partial(_gsum_body, n_chunks=n_chunks, n_iter=n_iter,
                          chunk=chunk, apply_relu=apply_relu, lanes=lanes),
        out_type=jax.ShapeDtypeStruct((n_out, D), table.dtype),
        mesh=mesh,
        scratch_types=[
            pltpu.VMEM((4, chunk), jnp.int32),
            pltpu.VMEM((4, chunk, D), table.dtype),
            pltpu.VMEM((chunk, D), table.dtype),
            pltpu.SemaphoreType.DMA,
        ],
        name=("gsum_relu" if apply_relu else "gsum"),
    )
    return kern(table, idxb)


def _mm_body(x_ref, w_ref, o_ref):
    acc = lax.dot_general(
        x_ref[...], w_ref[...], (((1,), (1,)), ((), ())),
        preferred_element_type=jnp.float32)
    o_ref[...] = acc.astype(o_ref.dtype)


def _mm(x, w, bm, out_dtype=jnp.float32):
    n, k = x.shape
    dout = w.shape[0]
    return pl.pallas_call(
        _mm_body,
        grid=(n // bm,),
        in_specs=[pl.BlockSpec((bm, k), lambda i: (i, 0)),
                  pl.BlockSpec((dout, k), lambda i: (0, 0))],
        out_specs=pl.BlockSpec((bm, dout), lambda i: (i, 0)),
        out_shape=jax.ShapeDtypeStruct((n, dout), out_dtype),
    )(x, w)


def _mm_add_relu_body(x_ref, w_ref, a_ref, o_ref):
    acc = lax.dot_general(x_ref[...], w_ref[...], (((1,), (1,)), ((), ())),
                          preferred_element_type=jnp.float32)
    acc = jnp.maximum(acc + a_ref[...].astype(jnp.float32), 0.0)
    o_ref[...] = acc.astype(o_ref.dtype)


def _mm_add_relu(x, w, add, bm, out_dtype=jnp.float32):
    n, k = x.shape
    dout = w.shape[0]
    return pl.pallas_call(
        _mm_add_relu_body,
        grid=(n // bm,),
        in_specs=[pl.BlockSpec((bm, k), lambda i: (i, 0)),
                  pl.BlockSpec((dout, k), lambda i: (0, 0)),
                  pl.BlockSpec((bm, dout), lambda i: (i, 0))],
        out_specs=pl.BlockSpec((bm, dout), lambda i: (i, 0)),
        out_shape=jax.ShapeDtypeStruct((n, dout), out_dtype),
    )(x, w, add)


def _out_body(af_ref, a_ref, wa_ref, wb_ref, b_ref, o_ref, *, bm, per):
    h = lax.dot_general(af_ref[...], wa_ref[...], (((1,), (1,)), ((), ())),
                        preferred_element_type=jnp.float32)
    h = h + lax.dot_general(a_ref[...], wb_ref[...], (((1,), (1,)), ((), ())),
                            preferred_element_type=jnp.float32)
    h = jnp.maximum(h + b_ref[...], 0.0)
    o_ref[...] = h.reshape(bm // per, per, D).sum(axis=1)


def _out_stage(af, a, wa, wb, b, bm, per):
    n = af.shape[0]
    ka = af.shape[1]
    n_mol_blk = bm // per
    return pl.pallas_call(
        functools.partial(_out_body, bm=bm, per=per),
        grid=(n // bm,),
        in_specs=[pl.BlockSpec((bm, ka), lambda i: (i, 0)),
                  pl.BlockSpec((bm, D), lambda i: (i, 0)),
                  pl.BlockSpec((D, ka), lambda i: (0, 0)),
                  pl.BlockSpec((D, D), lambda i: (0, 0)),
                  pl.BlockSpec((1, D), lambda i: (0, 0))],
        out_specs=pl.BlockSpec((n_mol_blk, D), lambda i: (i, 0)),
        out_shape=jax.ShapeDtypeStruct((n // per, D), jnp.float32),
    )(af, a, wa, wb, b)


def _pick(n, pref):
    return pref if n % pref == 0 else n


def kernel(atom_features, f_ini_atoms_bonds, atom_to_incoming_bonds, mapping,
           global_features, molecules_unbatch_key, W_i, W_h, W_o_w, W_o_b):
    n_bonds = f_ini_atoms_bonds.shape[0]
    n_atoms = atom_features.shape[0]
    afdim = atom_features.shape[1]

    cb = _pick(n_bonds, 128)
    ca = _pick(n_atoms, 80)
    bmb = _pick(n_bonds, 2000)
    bma = _pick(n_atoms, 2000)

    idxb_bonds = mapping.astype(jnp.int32).reshape(
        n_bonds // cb, cb, 4).transpose(0, 2, 1)
    idxb_atoms = atom_to_incoming_bonds.astype(jnp.int32).reshape(
        n_atoms // ca, ca, 4).transpose(0, 2, 1)

    inp = _mm(f_ini_atoms_bonds, W_i, bm=bmb)
    m1 = _gsum_sc(inp, idxb_bonds, n_bonds, cb, True)
    m2 = _gsum_sc(m1, idxb_bonds, n_bonds, cb, False)
    h = _mm_add_relu(m2, W_h, inp, bm=bmb)
    a = _gsum_sc(h, idxb_atoms, n_atoms, ca, False)

    wa = W_o_w[:, :afdim]
    wb = W_o_w[:, afdim:]
    mol_sum = _out_stage(atom_features, a, wa, wb,
                         W_o_b.reshape(1, D), bm=bma, per=25)
    mol = mol_sum / molecules_unbatch_key
    return jnp.concatenate([mol, global_features], axis=1)


# speedup vs baseline: 1.2840x; 1.2840x over previous
import functools

import jax
import jax.numpy as jnp
from jax import lax
from jax.experimental import pallas as pl
from jax.experimental.pallas import tpu as pltpu
from jax.experimental.pallas import tpu_sc as plsc

D = 128
LANES = 16
NW = 32


def _gsum_body(table, idxb, out, idx_v, rows_v, acc_v, sem0, sem1, *,
               n_chunks, n_iter, chunk, apply_relu):
    cid = lax.axis_index("c")
    sid = lax.axis_index("s")
    wid = sid * 2 + cid
    sems = (sem0, sem1)
    nv = D // LANES

    def fire(c, b):
        pltpu.sync_copy(idxb.at[c], idx_v.at[b])
        for j in range(4):
            pltpu.async_copy(table.at[idx_v.at[b, j]], rows_v.at[b, j],
                             sems[b])

    def drain(b):
        for j in range(4):
            pltpu.make_async_copy(table.at[idx_v.at[b, j]], rows_v.at[b, j],
                                  sems[b]).wait()

    def process(c, b):
        drain(b)

        def row_body(r, rc):
            for k in range(nv):
                s = pl.ds(k * LANES, LANES)
                vs = [rows_v[b, j, r, s] for j in range(4)]
                if apply_relu:
                    vs = [jnp.maximum(v, 0) for v in vs]
                acc_v[r, s] = (vs[0] + vs[1]) + (vs[2] + vs[3])
            return rc

        lax.fori_loop(0, chunk, row_body, 0)
        pltpu.sync_copy(acc_v, out.at[pl.ds(c * chunk, chunk)])

    fire(wid, 0)

    def pair_body(u, carry):
        for b in (0, 1):
            t = u * 2 + b
            c = t * NW + wid
            cn = c + NW

            @pl.when(cn < n_chunks)
            def _():
                fire(cn, 1 - b)

            @pl.when(c < n_chunks)
            def _():
                process(c, b)

        return carry

    lax.fori_loop(0, (n_iter + 1) // 2, pair_body, 0)


def _gsum_sc(table, idxb, n_out, chunk, apply_relu):
    n_chunks = n_out // chunk
    n_iter = (n_chunks + NW - 1) // NW
    mesh = plsc.VectorSubcoreMesh(core_axis_name="c", subcore_axis_name="s",
                                  num_cores=2, num_subcores=16)
    kern = pl.kernel(
        functools.partial(_gsum_body, n_chunks=n_chunks, n_iter=n_iter,
                          chunk=chunk, apply_relu=apply_relu),
        out_type=jax.ShapeDtypeStruct((n_out, D), table.dtype),
        mesh=mesh,
        scratch_types=[
            pltpu.VMEM((2, 4, chunk), jnp.int32),
            pltpu.VMEM((2, 4, chunk, D), table.dtype),
            pltpu.VMEM((chunk, D), table.dtype),
            pltpu.SemaphoreType.DMA,
            pltpu.SemaphoreType.DMA,
        ],
        name=("gsum_relu" if apply_relu else "gsum"),
    )
    return kern(table, idxb)


def _mm_body(x_ref, w_ref, o_ref):
    acc = lax.dot_general(
        x_ref[...], w_ref[...], (((1,), (1,)), ((), ())),
        preferred_element_type=jnp.float32)
    o_ref[...] = acc.astype(o_ref.dtype)


def _mm(x, w, bm, out_dtype=jnp.float32):
    n, k = x.shape
    dout = w.shape[0]
    return pl.pallas_call(
        _mm_body,
        grid=(n // bm,),
        in_specs=[pl.BlockSpec((bm, k), lambda i: (i, 0)),
                  pl.BlockSpec((dout, k), lambda i: (0, 0))],
        out_specs=pl.BlockSpec((bm, dout), lambda i: (i, 0)),
        out_shape=jax.ShapeDtypeStruct((n, dout), out_dtype),
    )(x, w)


def _mm_add_relu_body(x_ref, w_ref, a_ref, o_ref):
    acc = lax.dot_general(x_ref[...], w_ref[...], (((1,), (1,)), ((), ())),
                          preferred_element_type=jnp.float32)
    acc = jnp.maximum(acc + a_ref[...].astype(jnp.float32), 0.0)
    o_ref[...] = acc.astype(o_ref.dtype)


def _mm_add_relu(x, w, add, bm, out_dtype=jnp.float32):
    n, k = x.shape
    dout = w.shape[0]
    return pl.pallas_call(
        _mm_add_relu_body,
        grid=(n // bm,),
        in_specs=[pl.BlockSpec((bm, k), lambda i: (i, 0)),
                  pl.BlockSpec((dout, k), lambda i: (0, 0)),
                  pl.BlockSpec((bm, dout), lambda i: (i, 0))],
        out_specs=pl.BlockSpec((bm, dout), lambda i: (i, 0)),
        out_shape=jax.ShapeDtypeStruct((n, dout), out_dtype),
    )(x, w, add)


def _out_body(af_ref, a_ref, wa_ref, wb_ref, b_ref, o_ref, *, bm, per):
    h = lax.dot_general(af_ref[...], wa_ref[...], (((1,), (1,)), ((), ())),
                        preferred_element_type=jnp.float32)
    h = h + lax.dot_general(a_ref[...], wb_ref[...], (((1,), (1,)), ((), ())),
                            preferred_element_type=jnp.float32)
    h = jnp.maximum(h + b_ref[...], 0.0)
    o_ref[...] = h.reshape(bm // per, per, D).sum(axis=1)


def _out_stage(af, a, wa, wb, b, bm, per):
    n = af.shape[0]
    ka = af.shape[1]
    n_mol_blk = bm // per
    return pl.pallas_call(
        functools.partial(_out_body, bm=bm, per=per),
        grid=(n // bm,),
        in_specs=[pl.BlockSpec((bm, ka), lambda i: (i, 0)),
                  pl.BlockSpec((bm, D), lambda i: (i, 0)),
                  pl.BlockSpec((D, ka), lambda i: (0, 0)),
                  pl.BlockSpec((D, D), lambda i: (0, 0)),
                  pl.BlockSpec((1, D), lambda i: (0, 0))],
        out_specs=pl.BlockSpec((n_mol_blk, D), lambda i: (i, 0)),
        out_shape=jax.ShapeDtypeStruct((n // per, D), jnp.float32),
    )(af, a, wa, wb, b)


def _pick(n, pref):
    return pref if n % pref == 0 else n


def kernel(atom_features, f_ini_atoms_bonds, atom_to_incoming_bonds, mapping,
           global_features, molecules_unbatch_key, W_i, W_h, W_o_w, W_o_b):
    n_bonds = f_ini_atoms_bonds.shape[0]
    n_atoms = atom_features.shape[0]
    afdim = atom_features.shape[1]

    cb = _pick(n_bonds, 80)
    ca = _pick(n_atoms, 80)
    bmb = _pick(n_bonds, 2000)
    bma = _pick(n_atoms, 2000)

    idxb_bonds = mapping.astype(jnp.int32).reshape(
        n_bonds // cb, cb, 4).transpose(0, 2, 1)
    idxb_atoms = atom_to_incoming_bonds.astype(jnp.int32).reshape(
        n_atoms // ca, ca, 4).transpose(0, 2, 1)

    inp = _mm(f_ini_atoms_bonds, W_i, bm=bmb)
    m1 = _gsum_sc(inp, idxb_bonds, n_bonds, cb, True)
    m2 = _gsum_sc(m1, idxb_bonds, n_bonds, cb, False)
    h = _mm_add_relu(m2, W_h, inp, bm=bmb)
    a = _gsum_sc(h, idxb_atoms, n_atoms, ca, False)

    wa = W_o_w[:, :afdim]
    wb = W_o_w[:, afdim:]
    mol_sum = _out_stage(atom_features, a, wa, wb,
                         W_o_b.reshape(1, D), bm=bma, per=25)
    mol = mol_sum / molecules_unbatch_key
    return jnp.concatenate([mol, global_features], axis=1)


# speedup vs baseline: 1.3629x; 1.0615x over previous
import functools

import jax
import jax.numpy as jnp
from jax import lax
from jax.experimental import pallas as pl
from jax.experimental.pallas import tpu as pltpu
from jax.experimental.pallas import tpu_sc as plsc

D = 128
LANES = 16
NW = 32


def _gsum_body(*refs, n_chunks, n_iter, chunk, has_src):
    if has_src:
        (table, idxb, src, out, idx_v, rows_v, src_v, acc_v,
         sem0, sem1) = refs
    else:
        (table, idxb, out, idx_v, rows_v, acc_v, sem0, sem1) = refs
        src = src_v = None
    cid = lax.axis_index("c")
    sid = lax.axis_index("s")
    wid = sid * 2 + cid
    sems = (sem0, sem1)
    nv = D // LANES

    def fire(c, b):
        pltpu.sync_copy(idxb.at[c], idx_v.at[b])
        for j in range(4):
            pltpu.async_copy(table.at[idx_v.at[b, j]], rows_v.at[b, j],
                             sems[b])
        if has_src:
            pltpu.async_copy(src.at[pl.ds(c * chunk, chunk)], src_v.at[b],
                             sems[b])

    def drain(c, b):
        for j in range(4):
            pltpu.make_async_copy(table.at[idx_v.at[b, j]], rows_v.at[b, j],
                                  sems[b]).wait()
        if has_src:
            pltpu.make_async_copy(src.at[pl.ds(c * chunk, chunk)],
                                  src_v.at[b], sems[b]).wait()

    def process(c, b):
        drain(c, b)

        def row_body(r, rc):
            for k in range(nv):
                s = pl.ds(k * LANES, LANES)
                vs = [rows_v[b, j, r, s] for j in range(4)]
                v = (vs[0] + vs[1]) + (vs[2] + vs[3])
                if has_src:
                    v = jnp.maximum(v + src_v[b, r, s], 0)
                acc_v[r, s] = v
            return rc

        lax.fori_loop(0, chunk, row_body, 0)
        pltpu.sync_copy(acc_v, out.at[pl.ds(c * chunk, chunk)])

    fire(wid, 0)

    def pair_body(u, carry):
        for b in (0, 1):
            t = u * 2 + b
            c = t * NW + wid
            cn = c + NW

            @pl.when(cn < n_chunks)
            def _():
                fire(cn, 1 - b)

            @pl.when(c < n_chunks)
            def _():
                process(c, b)

        return carry

    lax.fori_loop(0, (n_iter + 1) // 2, pair_body, 0)


def _gsum_sc(table, idxb, n_out, chunk, src=None):
    n_chunks = n_out // chunk
    n_iter = (n_chunks + NW - 1) // NW
    has_src = src is not None
    mesh = plsc.VectorSubcoreMesh(core_axis_name="c", subcore_axis_name="s",
                                  num_cores=2, num_subcores=16)
    scratch = [
        pltpu.VMEM((2, 4, chunk), jnp.int32),
        pltpu.VMEM((2, 4, chunk, D), jnp.float32),
    ]
    if has_src:
        scratch.append(pltpu.VMEM((2, chunk, D), jnp.float32))
    scratch += [
        pltpu.VMEM((chunk, D), jnp.float32),
        pltpu.SemaphoreType.DMA,
        pltpu.SemaphoreType.DMA,
    ]
    kern = pl.kernel(
        functools.partial(_gsum_body, n_chunks=n_chunks, n_iter=n_iter,
                          chunk=chunk, has_src=has_src),
        out_type=jax.ShapeDtypeStruct((n_out, D), jnp.float32),
        mesh=mesh,
        scratch_types=scratch,
        name=("gsum_add_relu" if has_src else "gsum"),
    )
    return kern(table, idxb, src) if has_src else kern(table, idxb)


def _mm2_body(x_ref, wi_ref, wh_ref, inp_ref, z0_ref):
    inp = lax.dot_general(
        x_ref[...], wi_ref[...], (((1,), (1,)), ((), ())),
        preferred_element_type=jnp.float32)
    inp_ref[...] = inp
    z0_ref[...] = lax.dot_general(
        jnp.maximum(inp, 0.0), wh_ref[...], (((1,), (1,)), ((), ())),
        preferred_element_type=jnp.float32)


def _mm2(x, wi, wh, bm):
    n, k = x.shape
    return pl.pallas_call(
        _mm2_body,
        grid=(n // bm,),
        in_specs=[pl.BlockSpec((bm, k), lambda i: (i, 0)),
                  pl.BlockSpec((D, k), lambda i: (0, 0)),
                  pl.BlockSpec((D, D), lambda i: (0, 0))],
        out_specs=[pl.BlockSpec((bm, D), lambda i: (i, 0)),
                   pl.BlockSpec((bm, D), lambda i: (i, 0))],
        out_shape=[jax.ShapeDtypeStruct((n, D), jnp.float32),
                   jax.ShapeDtypeStruct((n, D), jnp.float32)],
    )(x, wi, wh)


def _out_body(af_ref, a_ref, wa_ref, wb_ref, b_ref, o_ref, *, bm, per):
    h = lax.dot_general(af_ref[...], wa_ref[...], (((1,), (1,)), ((), ())),
                        preferred_element_type=jnp.float32)
    h = h + lax.dot_general(a_ref[...], wb_ref[...], (((1,), (1,)), ((), ())),
                            preferred_element_type=jnp.float32)
    h = jnp.maximum(h + b_ref[...], 0.0)
    o_ref[...] = h.reshape(bm // per, per, D).sum(axis=1)


def _out_stage(af, a, wa, wb, b, bm, per):
    n = af.shape[0]
    ka = af.shape[1]
    n_mol_blk = bm // per
    return pl.pallas_call(
        functools.partial(_out_body, bm=bm, per=per),
        grid=(n // bm,),
        in_specs=[pl.BlockSpec((bm, ka), lambda i: (i, 0)),
                  pl.BlockSpec((bm, D), lambda i: (i, 0)),
                  pl.BlockSpec((D, ka), lambda i: (0, 0)),
                  pl.BlockSpec((D, D), lambda i: (0, 0)),
                  pl.BlockSpec((1, D), lambda i: (0, 0))],
        out_specs=pl.BlockSpec((n_mol_blk, D), lambda i: (i, 0)),
        out_shape=jax.ShapeDtypeStruct((n // per, D), jnp.float32),
    )(af, a, wa, wb, b)


def _pick(n, pref):
    return pref if n % pref == 0 else n


def kernel(atom_features, f_ini_atoms_bonds, atom_to_incoming_bonds, mapping,
           global_features, molecules_unbatch_key, W_i, W_h, W_o_w, W_o_b):
    n_bonds = f_ini_atoms_bonds.shape[0]
    n_atoms = atom_features.shape[0]
    afdim = atom_features.shape[1]

    cb = _pick(n_bonds, 80)
    ca = _pick(n_atoms, 80)
    bmb = _pick(n_bonds, 2000)
    bma = _pick(n_atoms, 2000)

    idxb_bonds = mapping.astype(jnp.int32).reshape(
        n_bonds // cb, cb, 4).transpose(0, 2, 1)
    idxb_atoms = atom_to_incoming_bonds.astype(jnp.int32).reshape(
        n_atoms // ca, ca, 4).transpose(0, 2, 1)

    inp, z0 = _mm2(f_ini_atoms_bonds, W_i, W_h, bm=bmb)
    zm = _gsum_sc(z0, idxb_bonds, n_bonds, cb)
    h = _gsum_sc(zm, idxb_bonds, n_bonds, cb, src=inp)
    a = _gsum_sc(h, idxb_atoms, n_atoms, ca)

    wa = W_o_w[:, :afdim]
    wb = W_o_w[:, afdim:]
    mol_sum = _out_stage(atom_features, a, wa, wb,
                         W_o_b.reshape(1, D), bm=bma, per=25)
    mol = mol_sum / molecules_unbatch_key
    return jnp.concatenate([mol, global_features], axis=1)
